# baseline (speedup 1.0000x reference)
import functools

import jax
import jax.numpy as jnp
from jax import lax
from jax.experimental import pallas as pl
from jax.experimental.pallas import tpu as pltpu
from jax.experimental.pallas import tpu_sc as plsc

N = 10000
E = 320000
D = 128
K = 27
BN_EPS = 1e-3

NC = 2
NS = 16
NW = NC * NS
C = 80
G = 126
GA = 128
PHASES = (32, 32, 32, 30)
PH_OFFS = (0, 32, 64, 96)
GMAX = 32
EPW = G * C
EP = NW * EPW
ACC_N = 10008
RPW = 632
RLAST = ACC_N - 15 * RPW


def _mm_body(x_ref, w_ref, o_ref):
    o_ref[0] = jnp.dot(x_ref[...], w_ref[0], preferred_element_type=jnp.float32)


def _mm(x, W):
    return pl.pallas_call(
        _mm_body,
        grid=(K,),
        in_specs=[
            pl.BlockSpec((N, D), lambda k: (0, 0)),
            pl.BlockSpec((1, D, D), lambda k: (k, 0, 0)),
        ],
        out_specs=pl.BlockSpec((1, N, D), lambda k: (k, 0, 0)),
        out_shape=jax.ShapeDtypeStruct((K, N, D), jnp.float32),
    )(x, W)


def _bn_relu(f, g_ref, be_ref):
    mu = jnp.mean(f, axis=0, keepdims=True)
    var = jnp.mean(f * f, axis=0, keepdims=True) - mu * mu
    return jnp.maximum((f - mu) * lax.rsqrt(var + BN_EPS) * g_ref[...] + be_ref[...], 0.0)


def _mid_body(p_ref, b_ref, g_ref, be_ref, w_ref, o_ref, h_ref):
    @pl.when(pl.program_id(0) == 0)
    def _():
        f = p_ref[0, :N, :] + p_ref[1, :N, :] + b_ref[...]
        h_ref[...] = _bn_relu(f, g_ref, be_ref)

    o_ref[0] = jnp.dot(h_ref[...], w_ref[0], preferred_element_type=jnp.float32)


def _mid(p, b1, g1, be1, W2):
    return pl.pallas_call(
        _mid_body,
        grid=(K,),
        in_specs=[
            pl.BlockSpec((NC, ACC_N, D), lambda k: (0, 0, 0)),
            pl.BlockSpec((1, D), lambda k: (0, 0)),
            pl.BlockSpec((1, D), lambda k: (0, 0)),
            pl.BlockSpec((1, D), lambda k: (0, 0)),
            pl.BlockSpec((1, D, D), lambda k: (k, 0, 0)),
        ],
        out_specs=pl.BlockSpec((1, N, D), lambda k: (k, 0, 0)),
        out_shape=jax.ShapeDtypeStruct((K, N, D), jnp.float32),
        scratch_shapes=[pltpu.VMEM((N, D), jnp.float32)],
    )(p, b1, g1, be1, W2)


def _final_body(p_ref, x_ref, b_ref, g_ref, be_ref, o_ref):
    f = p_ref[0, :N, :] + p_ref[1, :N, :] + b_ref[...]
    mu = jnp.mean(f, axis=0, keepdims=True)
    var = jnp.mean(f * f, axis=0, keepdims=True) - mu * mu
    bn = (f - mu) * lax.rsqrt(var + BN_EPS) * g_ref[...] + be_ref[...]
    o_ref[...] = jnp.maximum(bn + x_ref[...], 0.0)


def _final(p, x, b2, g2, be2):
    return pl.pallas_call(
        _final_body,
        out_shape=jax.ShapeDtypeStruct((N, D), jnp.float32),
    )(p, x, b2, g2, be2)


def _sc_gather_scatter(y_flat, gi_p, dst_p, zeros):
    mesh = plsc.VectorSubcoreMesh(core_axis_name="c", subcore_axis_name="s")

    @functools.partial(
        pl.kernel,
        out_type=jax.ShapeDtypeStruct((NC, ACC_N, D), jnp.float32),
        mesh=mesh,
        scratch_types=[
            pltpu.VMEM((GMAX, C), jnp.int32),
            pltpu.VMEM((GMAX, C), jnp.int32),
            pltpu.VMEM((C, D), jnp.float32),
            pltpu.VMEM((C, D), jnp.float32),
            pltpu.VMEM((C, D), jnp.float32),
            pltpu.VMEM((C, D), jnp.float32),
            pltpu.VMEM_SHARED((ACC_N, D), jnp.float32),
            pltpu.SemaphoreType.DMA,
            pltpu.SemaphoreType.DMA,
            pltpu.SemaphoreType.DMA,
            pltpu.SemaphoreType.DMA,
            pltpu.SemaphoreType.DMA,
            pltpu.SemaphoreType.DMA,
            pltpu.SemaphoreType.DMA,
            pltpu.SemaphoreType.DMA,
        ],
    )
    def k(y_hbm, gi_hbm, dst_hbm, z_hbm, out_hbm, gi_v, dst_v, msg0, msg1, msg2,
          msg3, acc, gs0, gs1, gs2, gs3, ss0, ss1, ss2, ss3):
        c = lax.axis_index("c")
        s = lax.axis_index("s")
        wid = c * NS + s
        msgs = (msg0, msg1, msg2, msg3)
        gsem = (gs0, gs1, gs2, gs3)
        ssem = (ss0, ss1, ss2, ss3)

        @pl.when(s < 15)
        def _():
            pltpu.sync_copy(z_hbm.at[pl.ds(s * RPW, RPW)],
                            acc.at[pl.ds(s * RPW, RPW)])

        @pl.when(s == 15)
        def _():
            pltpu.sync_copy(z_hbm.at[pl.ds(15 * RPW, RLAST)],
                            acc.at[pl.ds(15 * RPW, RLAST)])

        plsc.subcore_barrier()

        def gather(row, b):
            pltpu.async_copy(y_hbm.at[gi_v.at[row]], msgs[b], gsem[b])

        def gather_wait(row, b):
            pltpu.make_async_copy(y_hbm.at[gi_v.at[row]], msgs[b],
                                  gsem[b]).wait()

        def scatter(row, b):
            pltpu.async_copy(msgs[b], acc.at[dst_v.at[row]], ssem[b], add=True)

        def scatter_wait(row, b):
            pltpu.make_async_copy(msgs[b], acc.at[dst_v.at[row]],
                                  ssem[b]).wait()

        def one_iter(i_row, b, do_swait, bn, do_gather, g_row):
            gather_wait(i_row, b)
            scatter(i_row, b)
            if do_swait:
                scatter_wait(i_row, bn)
            if do_gather:
                gather(g_row, bn)

        for ph_off, GHp in zip(PH_OFFS, PHASES):
            pltpu.sync_copy(gi_hbm.at[wid, pl.ds(ph_off, GMAX)],
                            gi_v.at[pl.ds(0, GMAX)])
            pltpu.sync_copy(dst_hbm.at[wid, pl.ds(ph_off, GMAX)],
                            dst_v.at[pl.ds(0, GMAX)])
            gather(0, 0)
            gather(1, 1)
            rounds = (GHp - 4) // 4
            m_end = 2 + 4 * rounds
            for i in (0, 1):
                one_iter(i, i % 4, False, (i + 2) % 4, True, i + 2)

            def body(r, carry):
                for j in range(4):
                    i = 2 + 4 * r + j
                    b = (2 + j) % 4
                    bn = (2 + j + 2) % 4
                    one_iter(i, b, True, bn, True, i + 2)
                return carry

            lax.fori_loop(0, rounds, body, 0)
            for i in range(m_end, GHp):
                one_iter(i, i % 4, True, (i + 2) % 4, i + 2 < GHp, i + 2)
            scatter_wait(0, (GHp - 2) % 4)
            scatter_wait(0, (GHp - 1) % 4)

        plsc.subcore_barrier()

        @pl.when(s < 15)
        def _():
            pltpu.sync_copy(acc.at[pl.ds(s * RPW, RPW)],
                            out_hbm.at[c, pl.ds(s * RPW, RPW)])

        @pl.when(s == 15)
        def _():
            pltpu.sync_copy(acc.at[pl.ds(15 * RPW, RLAST)],
                            out_hbm.at[c, pl.ds(15 * RPW, RLAST)])

    return k(y_flat, gi_p, dst_p, zeros)


def kernel(x, edge_index, kernel_idx, W1, b1, g1, be1, W2, b2, g2, be2):
    src = edge_index[0].astype(jnp.int32)
    dst = edge_index[1].astype(jnp.int32)
    ki = kernel_idx.astype(jnp.int32)

    pad = EP - E
    gi = ki * N + src
    gi_p = jnp.pad(
        jnp.concatenate([gi, jnp.zeros((pad,), jnp.int32)]).reshape(NW, G, C),
        ((0, 0), (0, GA - G), (0, 0)))
    dst_p = jnp.pad(
        jnp.concatenate([dst, jnp.full((pad,), N, jnp.int32)]).reshape(NW, G, C),
        ((0, 0), (0, GA - G), (0, 0)))
    zeros = jnp.zeros((ACC_N, D), jnp.float32)

    b1r, g1r, be1r = b1.reshape(1, D), g1.reshape(1, D), be1.reshape(1, D)
    b2r, g2r, be2r = b2.reshape(1, D), g2.reshape(1, D), be2.reshape(1, D)

    y1 = _mm(x, W1).reshape(K * N, D)
    p1 = _sc_gather_scatter(y1, gi_p, dst_p, zeros)
    y2 = _mid(p1, b1r, g1r, be1r, W2).reshape(K * N, D)
    p2 = _sc_gather_scatter(y2, gi_p, dst_p, zeros)
    return _final(p2, x, b2r, g2r, be2r)


# speedup vs baseline: 1.1299x; 1.1299x over previous
import functools

import jax
import jax.numpy as jnp
from jax import lax
from jax.experimental import pallas as pl
from jax.experimental.pallas import tpu as pltpu
from jax.experimental.pallas import tpu_sc as plsc

N = 10000
E = 320000
D = 128
K = 27
BN_EPS = 1e-3

NC = 2
NS = 16
NW = NC * NS
C = 104
G = 97
GA = 104
PHASES = (40, 40, 17)
GMAX = 40
EPW = G * C
EP = NW * EPW
ACC_N = 10008
RPW = 632
RLAST = ACC_N - 15 * RPW


def _mm_body(x_ref, w_ref, o_ref):
    o_ref[0] = jnp.dot(x_ref[...], w_ref[0], preferred_element_type=jnp.float32)


def _mm(x, W):
    return pl.pallas_call(
        _mm_body,
        grid=(K,),
        in_specs=[
            pl.BlockSpec((N, D), lambda k: (0, 0)),
            pl.BlockSpec((1, D, D), lambda k: (k, 0, 0)),
        ],
        out_specs=pl.BlockSpec((1, N, D), lambda k: (k, 0, 0)),
        out_shape=jax.ShapeDtypeStruct((K, N, D), jnp.float32),
    )(x, W)


def _bn_relu(f, g_ref, be_ref):
    mu = jnp.mean(f, axis=0, keepdims=True)
    var = jnp.mean(f * f, axis=0, keepdims=True) - mu * mu
    return jnp.maximum((f - mu) * lax.rsqrt(var + BN_EPS) * g_ref[...] + be_ref[...], 0.0)


def _mid_body(p_ref, b_ref, g_ref, be_ref, w_ref, o_ref, h_ref):
    @pl.when(pl.program_id(0) == 0)
    def _():
        f = p_ref[0, :N, :] + p_ref[1, :N, :] + b_ref[...]
        h_ref[...] = _bn_relu(f, g_ref, be_ref)

    o_ref[0] = jnp.dot(h_ref[...], w_ref[0], preferred_element_type=jnp.float32)


def _mid(p, b1, g1, be1, W2):
    return pl.pallas_call(
        _mid_body,
        grid=(K,),
        in_specs=[
            pl.BlockSpec((NC, ACC_N, D), lambda k: (0, 0, 0)),
            pl.BlockSpec((1, D), lambda k: (0, 0)),
            pl.BlockSpec((1, D), lambda k: (0, 0)),
            pl.BlockSpec((1, D), lambda k: (0, 0)),
            pl.BlockSpec((1, D, D), lambda k: (k, 0, 0)),
        ],
        out_specs=pl.BlockSpec((1, N, D), lambda k: (k, 0, 0)),
        out_shape=jax.ShapeDtypeStruct((K, N, D), jnp.float32),
        scratch_shapes=[pltpu.VMEM((N, D), jnp.float32)],
    )(p, b1, g1, be1, W2)


def _final_body(p_ref, x_ref, b_ref, g_ref, be_ref, o_ref):
    f = p_ref[0, :N, :] + p_ref[1, :N, :] + b_ref[...]
    mu = jnp.mean(f, axis=0, keepdims=True)
    var = jnp.mean(f * f, axis=0, keepdims=True) - mu * mu
    bn = (f - mu) * lax.rsqrt(var + BN_EPS) * g_ref[...] + be_ref[...]
    o_ref[...] = jnp.maximum(bn + x_ref[...], 0.0)


def _final(p, x, b2, g2, be2):
    return pl.pallas_call(
        _final_body,
        out_shape=jax.ShapeDtypeStruct((N, D), jnp.float32),
    )(p, x, b2, g2, be2)


def _sc_gather_scatter(y_flat, gi_p, dst_p, zeros):
    mesh = plsc.VectorSubcoreMesh(core_axis_name="c", subcore_axis_name="s")

    @functools.partial(
        pl.kernel,
        out_type=jax.ShapeDtypeStruct((NC, ACC_N, D), jnp.float32),
        mesh=mesh,
        scratch_types=[
            pltpu.VMEM((GMAX, C), jnp.int32),
            pltpu.VMEM((GMAX, C), jnp.int32),
            pltpu.VMEM((C, D), jnp.float32),
            pltpu.VMEM((C, D), jnp.float32),
            pltpu.VMEM((C, D), jnp.float32),
            pltpu.VMEM_SHARED((ACC_N, D), jnp.float32),
            pltpu.SemaphoreType.DMA,
            pltpu.SemaphoreType.DMA,
            pltpu.SemaphoreType.DMA,
            pltpu.SemaphoreType.DMA,
            pltpu.SemaphoreType.DMA,
            pltpu.SemaphoreType.DMA,
        ],
    )
    def k(y_hbm, gi_hbm, dst_hbm, z_hbm, out_hbm, gi_v, dst_v, msg0, msg1, msg2,
          acc, gs0, gs1, gs2, ss0, ss1, ss2):
        c = lax.axis_index("c")
        s = lax.axis_index("s")
        wid = c * NS + s
        msgs = (msg0, msg1, msg2)
        gsem = (gs0, gs1, gs2)
        ssem = (ss0, ss1, ss2)

        @pl.when(s < 15)
        def _():
            pltpu.sync_copy(z_hbm.at[pl.ds(s * RPW, RPW)],
                            acc.at[pl.ds(s * RPW, RPW)])

        @pl.when(s == 15)
        def _():
            pltpu.sync_copy(z_hbm.at[pl.ds(15 * RPW, RLAST)],
                            acc.at[pl.ds(15 * RPW, RLAST)])

        plsc.subcore_barrier()

        def gather(row, b):
            pltpu.async_copy(y_hbm.at[gi_v.at[row]], msgs[b], gsem[b])

        def gather_wait(row, b):
            pltpu.make_async_copy(y_hbm.at[gi_v.at[row]], msgs[b],
                                  gsem[b]).wait()

        def scatter(row, b):
            pltpu.async_copy(msgs[b], acc.at[dst_v.at[row]], ssem[b], add=True)

        def scatter_wait(row, b):
            pltpu.make_async_copy(msgs[b], acc.at[dst_v.at[row]],
                                  ssem[b]).wait()

        def one_iter(i_row, b, do_swait, bn, do_gather, g_row):
            gather_wait(i_row, b)
            scatter(i_row, b)
            if do_swait:
                scatter_wait(i_row, bn)
            if do_gather:
                gather(g_row, bn)

        for ph_off, GHp in zip((0, 40, 80), PHASES):
            GHs = ((GHp + 7) // 8) * 8
            pltpu.sync_copy(gi_hbm.at[wid, pl.ds(ph_off, GHs)],
                            gi_v.at[pl.ds(0, GHs)])
            pltpu.sync_copy(dst_hbm.at[wid, pl.ds(ph_off, GHs)],
                            dst_v.at[pl.ds(0, GHs)])
            gather(0, 0)
            gather(1, 1)
            rounds = (GHp - 4) // 3
            m_end = 2 + 3 * rounds
            for i in (0, 1):
                one_iter(i, i % 3, i >= 1, (i + 2) % 3, True, i + 2)

            def body(r, carry):
                for j in range(3):
                    i = 2 + 3 * r + j
                    b = (2 + j) % 3
                    bn = (2 + j + 2) % 3
                    one_iter(i, b, True, bn, True, i + 2)
                return carry

            lax.fori_loop(0, rounds, body, 0)
            for i in range(m_end, GHp):
                one_iter(i, i % 3, True, (i + 2) % 3, i + 2 < GHp, i + 2)
            scatter_wait(0, (GHp - 1) % 3)

        plsc.subcore_barrier()

        @pl.when(s < 15)
        def _():
            pltpu.sync_copy(acc.at[pl.ds(s * RPW, RPW)],
                            out_hbm.at[c, pl.ds(s * RPW, RPW)])

        @pl.when(s == 15)
        def _():
            pltpu.sync_copy(acc.at[pl.ds(15 * RPW, RLAST)],
                            out_hbm.at[c, pl.ds(15 * RPW, RLAST)])

    return k(y_flat, gi_p, dst_p, zeros)


def kernel(x, edge_index, kernel_idx, W1, b1, g1, be1, W2, b2, g2, be2):
    src = edge_index[0].astype(jnp.int32)
    dst = edge_index[1].astype(jnp.int32)
    ki = kernel_idx.astype(jnp.int32)

    pad = EP - E
    gi = ki * N + src
    gi_p = jnp.pad(
        jnp.concatenate([gi, jnp.zeros((pad,), jnp.int32)]).reshape(NW, G, C),
        ((0, 0), (0, GA - G), (0, 0)))
    dst_p = jnp.pad(
        jnp.concatenate([dst, jnp.full((pad,), N, jnp.int32)]).reshape(NW, G, C),
        ((0, 0), (0, GA - G), (0, 0)))
    zeros = jnp.zeros((ACC_N, D), jnp.float32)

    b1r, g1r, be1r = b1.reshape(1, D), g1.reshape(1, D), be1.reshape(1, D)
    b2r, g2r, be2r = b2.reshape(1, D), g2.reshape(1, D), be2.reshape(1, D)

    y1 = _mm(x, W1).reshape(K * N, D)
    p1 = _sc_gather_scatter(y1, gi_p, dst_p, zeros)
    y2 = _mid(p1, b1r, g1r, be1r, W2).reshape(K * N, D)
    p2 = _sc_gather_scatter(y2, gi_p, dst_p, zeros)
    return _final(p2, x, b2r, g2r, be2r)
